# baseline (speedup 1.0000x reference)
import jax
import jax.numpy as jnp
from jax import lax
from jax.experimental import pallas as pl
from jax.experimental.pallas import tpu as pltpu
from jax.experimental.pallas import tpu_sc as plsc

NUM_CORES = 2
NUM_SUBCORES = 16
LANES = 16
NUM_WORKERS = NUM_CORES * NUM_SUBCORES

BATCH = 16384
FACTORS = 32
B_PER_W = BATCH // NUM_WORKERS
GROUP = 8
N_GROUPS = B_PER_W // GROUP
PAD = LANES


def _sc_body(user_hbm, item_hbm, utabT, itabT, out_hbm,
             uidx_v, iidx_v, ustage, istage, out_v, usem, isem):
    wid = lax.axis_index("s") * NUM_CORES + lax.axis_index("c")
    base = wid * B_PER_W

    pltpu.sync_copy(user_hbm.at[pl.ds(base, B_PER_W)], uidx_v.at[pl.ds(0, B_PER_W)])
    pltpu.sync_copy(item_hbm.at[pl.ds(base, B_PER_W)], iidx_v.at[pl.ds(0, B_PER_W)])

    lane = lax.iota(jnp.int32, LANES)
    slotv = lane & (GROUP - 1)
    lomask = lane < GROUP

    def group(g, carry):
        uvec = uidx_v[pl.ds(g * GROUP, LANES)]
        ivec = iidx_v[pl.ds(g * GROUP, LANES)]
        handles = []
        for j in range(GROUP):
            ucs = pl.multiple_of((uvec[j] >> 7) * 128, 128)
            ics = pl.multiple_of((ivec[j] >> 7) * 128, 128)
            for t in range(4):
                handles.append(pltpu.async_copy(
                    utabT.at[pl.ds(t * 8, 8), pl.ds(ucs, 128)],
                    ustage.at[j, pl.ds(t * 8, 8)], usem))
                handles.append(pltpu.async_copy(
                    itabT.at[pl.ds(t * 8, 8), pl.ds(ics, 128)],
                    istage.at[j, pl.ds(t * 8, 8)], isem))
        for h in handles:
            h.wait()

        ucol = uvec & 127
        icol = ivec & 127
        acc = jnp.zeros((LANES,), jnp.float32)
        for f in range(FACTORS):
            fv = jnp.full((LANES,), f, jnp.int32)
            uval = plsc.load_gather(ustage, [slotv, fv, ucol])
            ival = plsc.load_gather(istage, [slotv, fv, icol])
            acc = acc + uval * ival
        plsc.store_compressed(out_v.at[pl.ds(g * GROUP, LANES)], acc,
                              mask=lomask)
        return carry

    lax.fori_loop(0, N_GROUPS, group, 0)

    pltpu.sync_copy(out_v.at[pl.ds(0, B_PER_W)],
                    out_hbm.at[pl.ds(base, B_PER_W)])


@jax.jit
def kernel(user, item, user_table, item_table):
    call = pl.kernel(
        _sc_body,
        out_type=jax.ShapeDtypeStruct((BATCH,), jnp.float32),
        mesh=plsc.VectorSubcoreMesh(
            core_axis_name="c", subcore_axis_name="s",
            num_cores=NUM_CORES, num_subcores=NUM_SUBCORES),
        compiler_params=pltpu.CompilerParams(
            needs_layout_passes=False, use_tc_tiling_on_sc=True),
        scratch_types=[
            pltpu.VMEM((B_PER_W + PAD,), jnp.int32),
            pltpu.VMEM((B_PER_W + PAD,), jnp.int32),
            pltpu.VMEM((GROUP, FACTORS, 128), jnp.float32),
            pltpu.VMEM((GROUP, FACTORS, 128), jnp.float32),
            pltpu.VMEM((B_PER_W + PAD,), jnp.float32),
            pltpu.SemaphoreType.DMA,
            pltpu.SemaphoreType.DMA,
        ],
    )
    return call(user.astype(jnp.int32), item.astype(jnp.int32),
                user_table.T, item_table.T)


# speedup vs baseline: 1.0491x; 1.0491x over previous
import jax
import jax.numpy as jnp
from jax import lax
from jax.experimental import pallas as pl
from jax.experimental.pallas import tpu as pltpu
from jax.experimental.pallas import tpu_sc as plsc

NUM_CORES = 2
NUM_SUBCORES = 16
LANES = 16
NUM_WORKERS = NUM_CORES * NUM_SUBCORES

BATCH = 16384
FACTORS = 32
B_PER_W = BATCH // NUM_WORKERS
GROUP = 4
N_GROUPS = B_PER_W // GROUP
PAD = LANES


def _sc_body(user_hbm, item_hbm, utabT, itabT, out_hbm,
             uidx_v, iidx_v, ustageA, istageA, ustageB, istageB, out_v,
             semA, semB):
    wid = lax.axis_index("s") * NUM_CORES + lax.axis_index("c")
    base = wid * B_PER_W

    pltpu.sync_copy(user_hbm.at[pl.ds(base, B_PER_W)],
                    uidx_v.at[pl.ds(0, B_PER_W)])
    pltpu.sync_copy(item_hbm.at[pl.ds(base, B_PER_W)],
                    iidx_v.at[pl.ds(0, B_PER_W)])

    lane = lax.iota(jnp.int32, LANES)
    slotv = lane & (GROUP - 1)
    lomask = lane < GROUP

    def issue(g, uset, iset, sem):
        uvec = uidx_v[pl.ds(g * GROUP, LANES)]
        ivec = iidx_v[pl.ds(g * GROUP, LANES)]
        handles = []
        for j in range(GROUP):
            ucs = pl.multiple_of((uvec[j] >> 7) * 128, 128)
            ics = pl.multiple_of((ivec[j] >> 7) * 128, 128)
            handles.append(pltpu.async_copy(
                utabT.at[:, pl.ds(ucs, 128)], uset.at[j], sem))
            handles.append(pltpu.async_copy(
                itabT.at[:, pl.ds(ics, 128)], iset.at[j], sem))
        return handles

    def drain(uset, iset, sem):
        for j in range(GROUP):
            pltpu.make_async_copy(
                utabT.at[:, pl.ds(0, 128)], uset.at[j], sem).wait()
            pltpu.make_async_copy(
                itabT.at[:, pl.ds(0, 128)], iset.at[j], sem).wait()

    def reduce_group(g, uset, iset):
        uvec = uidx_v[pl.ds(g * GROUP, LANES)]
        ivec = iidx_v[pl.ds(g * GROUP, LANES)]
        ucol = uvec & 127
        icol = ivec & 127
        acc = jnp.zeros((LANES,), jnp.float32)
        for f in range(FACTORS):
            fv = jnp.full((LANES,), f, jnp.int32)
            uval = plsc.load_gather(uset, [slotv, fv, ucol])
            ival = plsc.load_gather(iset, [slotv, fv, icol])
            acc = acc + uval * ival
        plsc.store_compressed(out_v.at[pl.ds(g * GROUP, LANES)], acc,
                              mask=lomask)

    issue(0, ustageA, istageA, semA)

    def pair(p, carry):
        gA = 2 * p
        gB = 2 * p + 1
        issue(gB, ustageB, istageB, semB)
        drain(ustageA, istageA, semA)
        reduce_group(gA, ustageA, istageA)
        gA2 = jnp.minimum(gA + 2, N_GROUPS - 1)
        issue(gA2, ustageA, istageA, semA)
        drain(ustageB, istageB, semB)
        reduce_group(gB, ustageB, istageB)
        return carry

    lax.fori_loop(0, N_GROUPS // 2, pair, 0)
    drain(ustageA, istageA, semA)

    pltpu.sync_copy(out_v.at[pl.ds(0, B_PER_W)],
                    out_hbm.at[pl.ds(base, B_PER_W)])


@jax.jit
def kernel(user, item, user_table, item_table):
    call = pl.kernel(
        _sc_body,
        out_type=jax.ShapeDtypeStruct((BATCH,), jnp.float32),
        mesh=plsc.VectorSubcoreMesh(
            core_axis_name="c", subcore_axis_name="s",
            num_cores=NUM_CORES, num_subcores=NUM_SUBCORES),
        compiler_params=pltpu.CompilerParams(
            needs_layout_passes=False, use_tc_tiling_on_sc=True),
        scratch_types=[
            pltpu.VMEM((B_PER_W + PAD,), jnp.int32),
            pltpu.VMEM((B_PER_W + PAD,), jnp.int32),
            pltpu.VMEM((GROUP, FACTORS, 128), jnp.float32),
            pltpu.VMEM((GROUP, FACTORS, 128), jnp.float32),
            pltpu.VMEM((GROUP, FACTORS, 128), jnp.float32),
            pltpu.VMEM((GROUP, FACTORS, 128), jnp.float32),
            pltpu.VMEM((B_PER_W + PAD,), jnp.float32),
            pltpu.SemaphoreType.DMA,
            pltpu.SemaphoreType.DMA,
        ],
    )
    return call(user.astype(jnp.int32), item.astype(jnp.int32),
                user_table.T, item_table.T)
